# Initial kernel scaffold; baseline (speedup 1.0000x reference)
import jax
import jax.numpy as jnp
from jax.experimental import pallas as pl


def kernel(x, edge_index_onset, edge_index_consecutive, pot_edges, pot_chord_edges, batch, onsets, durations, pitches, onset_beat, duration_beat, ts_beats, params):
    raise NotImplementedError("write your pallas kernel here")


# speedup vs baseline: 1.3730x; 1.3730x over previous
import functools

import jax
import jax.numpy as jnp
from jax import lax
from jax.experimental import pallas as pl
from jax.experimental.pallas import tpu as pltpu
from jax.experimental.pallas import tpu_sc as plsc

_N = 50000
_H = 128
_EXT = 256
_E = 400000
_EP = 400000

_CHUNK = 6272
_NCHUNK_PER_CORE = 4
_NPAD = 8 * _CHUNK
_ACC_ROWS = _CHUNK + 16
_STRIP = _CHUNK // 16
_STRIP_PIECES = [(0, 128), (128, 128), (256, 128), (384, 8)]

_E_PAD = 401408
_EPT = _E_PAD // 16
_SB = 64
_NBATCH = _EPT // _SB

_EP_PAD = 401408
_EPTD = _EP_PAD // 32
_NBD = _EPTD // 128

_NB = 512
_NGRID = _NPAD // _NB
_EB = 4096
_EGRID = _EP_PAD // _EB


def _ln(v, g, b):
    m = jnp.mean(v, axis=-1, keepdims=True)
    var = jnp.mean((v - m) ** 2, axis=-1, keepdims=True)
    return (v - m) * lax.rsqrt(var + 1e-5) * g + b


def _ones_tail(nrows):
    one = jnp.ones((nrows, 1), jnp.float32)
    return jnp.concatenate([one, jnp.zeros((nrows, _EXT - _H - 1), jnp.float32)], axis=1)


def _k1_body(x_ref, w_ref, b_ref, g_ref, bb_ref, o_ref):
    v = jnp.dot(x_ref[...], w_ref[...], preferred_element_type=jnp.float32) + b_ref[...]
    v = _ln(jnp.maximum(v, 0.0), g_ref[...], bb_ref[...])
    o_ref[...] = jnp.concatenate([v, _ones_tail(v.shape[0])], axis=1)


def _k2a_body(son_ref, sco_ref, h_ref, wl0_ref, wl1_ref, wrs_ref, bs_ref,
              a_ref, st_ref):
    i = pl.program_id(0)
    mon = son_ref[:, :_H] / jnp.maximum(son_ref[:, _H:_H + 1], 1.0)
    mco = sco_ref[:, :_H] / jnp.maximum(sco_ref[:, _H:_H + 1], 1.0)
    h = h_ref[:, :_H]
    pre = (jnp.dot(mon, wl0_ref[...], preferred_element_type=jnp.float32)
           + jnp.dot(mco, wl1_ref[...], preferred_element_type=jnp.float32)
           + jnp.dot(h, wrs_ref[...], preferred_element_type=jnp.float32)
           + bs_ref[...])
    a = jnp.maximum(pre, 0.0)
    a_ref[...] = a
    rid = i * _NB + lax.broadcasted_iota(jnp.int32, (_NB, 1), 0)
    am = jnp.where(rid < _N, a, 0.0)
    contrib = jnp.concatenate(
        [jnp.sum(am, axis=0, keepdims=True),
         jnp.sum(am * am, axis=0, keepdims=True),
         jnp.zeros((6, _H), jnp.float32)], axis=0)

    @pl.when(i == 0)
    def _():
        st_ref[...] = jnp.zeros_like(st_ref)

    st_ref[...] += contrib


def _k2b_body(a_ref, st_ref, w_ref, b_ref, ms_ref, o_ref):
    a = a_ref[...]
    st = st_ref[...]
    m = st[0:1] * (1.0 / _N)
    s2 = st[1:2] * (1.0 / _N)
    ms = ms_ref[...]
    var = s2 - (2.0 * ms - ms * ms) * (m * m)
    v = (a - ms * m) * lax.rsqrt(var + 1e-5) * w_ref[...] + b_ref[...]
    o_ref[...] = jnp.concatenate([v, _ones_tail(v.shape[0])], axis=1)


def _k3_body(son_ref, sco_ref, h1_ref, ft_ref, wl0_ref, wl1_ref, wrs_ref, bs_ref,
             sw1_ref, sb1_ref, sg_ref, sbb_ref, sw2_ref, sb2_ref, at_ref, bt_ref,
             hid_ref, stf_ref, p_ref, q_ref):
    mon = son_ref[:, :_H] / jnp.maximum(son_ref[:, _H:_H + 1], 1.0)
    mco = sco_ref[:, :_H] / jnp.maximum(sco_ref[:, _H:_H + 1], 1.0)
    h = h1_ref[:, :_H]
    hid = (jnp.dot(mon, wl0_ref[...], preferred_element_type=jnp.float32)
           + jnp.dot(mco, wl1_ref[...], preferred_element_type=jnp.float32)
           + jnp.dot(h, wrs_ref[...], preferred_element_type=jnp.float32)
           + bs_ref[...])
    hid_ref[...] = hid
    t = jnp.maximum(jnp.dot(hid, sw1_ref[...], preferred_element_type=jnp.float32) + sb1_ref[...], 0.0)
    t = _ln(t, sg_ref[...], sbb_ref[...])
    stf_ref[...] = jax.nn.sigmoid(jnp.dot(t, sw2_ref[...], preferred_element_type=jnp.float32) + sb2_ref[...])
    p = jnp.dot(hid, at_ref[...], preferred_element_type=jnp.float32)
    q = jnp.dot(hid, bt_ref[...], preferred_element_type=jnp.float32)
    f = ft_ref[...]
    offb = f[:, 3:4] + f[:, 4:5]
    off = f[:, 0:1] + f[:, 1:2]
    pit = f[:, 2:3]
    z1 = jnp.zeros_like(pit)
    nz = _EXT - _H
    ptail = jnp.concatenate([offb, off, pit] + [z1] * (nz - 3), axis=1)
    qtail = jnp.concatenate([z1] * 3 + [f[:, 3:4], f[:, 5:6], f[:, 0:1], pit] + [z1] * (nz - 7), axis=1)
    p_ref[...] = jnp.concatenate([p, ptail], axis=1)
    q_ref[...] = jnp.concatenate([q, qtail], axis=1)


def _k4_body(r_ref, c_ref, o_ref):
    r = r_ref[...]
    c = c_ref[...]
    z0 = r[:, :_H]
    offb = r[:, _H]
    off = r[:, _H + 1]
    pit_r = r[:, _H + 2]
    onb = r[:, _H + 3]
    ts = r[:, _H + 4]
    ons = r[:, _H + 5]
    pit_c = r[:, _H + 6]
    os1 = 1.0 - jnp.tanh((onb - offb) / ts)
    oh = (ons == off).astype(jnp.float32)
    ps = jnp.abs(pit_c - pit_r) * (1.0 / 127.0)
    z = (z0 + os1[:, None] * c[0:1] + oh[:, None] * c[1:2]
         + ps[:, None] * c[2:3] + c[3:4])
    z = jnp.maximum(z, 0.0)
    z = _ln(z, c[4:5], c[5:6])
    out = jnp.sum(z * c[6:7], axis=-1) + c[7, 0]
    o_ref[...] = out.reshape(_EB // 128, 128)


def _node_spec(width):
    return pl.BlockSpec((_NB, width), lambda i: (i, 0))


def _full_spec(r, cdim):
    return pl.BlockSpec((r, cdim), lambda i: (0, 0))


_k1 = pl.pallas_call(
    _k1_body,
    grid=(_NGRID,),
    in_specs=[_node_spec(_H), _full_spec(_H, _H), _full_spec(1, _H),
              _full_spec(1, _H), _full_spec(1, _H)],
    out_specs=_node_spec(_EXT),
    out_shape=jax.ShapeDtypeStruct((_NPAD, _EXT), jnp.float32),
)

_k2a = pl.pallas_call(
    _k2a_body,
    grid=(_NGRID,),
    in_specs=[_node_spec(_EXT), _node_spec(_EXT), _node_spec(_EXT),
              _full_spec(_H, _H), _full_spec(_H, _H), _full_spec(_H, _H),
              _full_spec(1, _H)],
    out_specs=[_node_spec(_H), _full_spec(8, _H)],
    out_shape=[jax.ShapeDtypeStruct((_NPAD, _H), jnp.float32),
               jax.ShapeDtypeStruct((8, _H), jnp.float32)],
)

_k2b = pl.pallas_call(
    _k2b_body,
    grid=(_NGRID,),
    in_specs=[_node_spec(_H), _full_spec(8, _H), _full_spec(1, _H),
              _full_spec(1, _H), _full_spec(1, _H)],
    out_specs=_node_spec(_EXT),
    out_shape=jax.ShapeDtypeStruct((_NPAD, _EXT), jnp.float32),
)

_k3 = pl.pallas_call(
    _k3_body,
    grid=(_NGRID,),
    in_specs=[_node_spec(_EXT), _node_spec(_EXT), _node_spec(_EXT),
              _node_spec(8),
              _full_spec(_H, _H), _full_spec(_H, _H), _full_spec(_H, _H),
              _full_spec(1, _H),
              _full_spec(_H, _H), _full_spec(1, _H), _full_spec(1, _H),
              _full_spec(1, _H), _full_spec(_H, _H), _full_spec(1, _H),
              _full_spec(_H, _H), _full_spec(_H, _H)],
    out_specs=[_node_spec(_H), _node_spec(_H), _node_spec(_EXT), _node_spec(_EXT)],
    out_shape=[jax.ShapeDtypeStruct((_NPAD, _H), jnp.float32),
               jax.ShapeDtypeStruct((_NPAD, _H), jnp.float32),
               jax.ShapeDtypeStruct((_NPAD, _EXT), jnp.float32),
               jax.ShapeDtypeStruct((_NPAD, _EXT), jnp.float32)],
)

_k4 = pl.pallas_call(
    _k4_body,
    grid=(_EGRID,),
    in_specs=[pl.BlockSpec((_EB, _EXT), lambda i: (i, 0)), _full_spec(8, _H)],
    out_specs=pl.BlockSpec((_EB // 128, 128), lambda i: (i, 0)),
    out_shape=jax.ShapeDtypeStruct((_EP_PAD // 128, 128), jnp.float32),
)


_MESH = plsc.VectorSubcoreMesh(core_axis_name="c", subcore_axis_name="s")


def _segsum_body(h_hbm, sa_hbm, da_hbm, sb_hbm, db_hbm, z_hbm, outa, outb,
                 acc, srcb, dstb, lidx, rowbuf, sem):
    c = lax.axis_index("c")
    s = lax.axis_index("s")
    for src_hbm, dst_hbm, out_hbm in ((sa_hbm, da_hbm, outa), (sb_hbm, db_hbm, outb)):
        for k in range(_NCHUNK_PER_CORE):
            base = (_NCHUNK_PER_CORE * c + k) * _CHUNK
            pltpu.sync_copy(z_hbm, acc.at[pl.ds(s * _STRIP, _STRIP)])

            @pl.when(s == 0)
            def _():
                pltpu.sync_copy(z_hbm.at[pl.ds(0, 16)], acc.at[pl.ds(_CHUNK, 16)])

            plsc.subcore_barrier()

            def batch(b, _):
                off_e = s * _EPT + b * _SB
                pltpu.sync_copy(dst_hbm.at[pl.ds(off_e, _SB)], dstb)
                pltpu.sync_copy(src_hbm.at[pl.ds(off_e, _SB)], srcb)
                for g in range(_SB // 16):
                    d = dstb[pl.ds(g * 16, 16)]
                    inb = (d >= base) & (d < base + _CHUNK)
                    loc = jnp.where(inb, d - base, _CHUNK)
                    lidx[0, pl.ds(g * 16, 16)] = loc
                pltpu.async_copy(h_hbm.at[srcb], rowbuf, sem).wait()
                pltpu.async_copy(rowbuf, acc.at[lidx.at[0]], sem, add=True).wait()
                return 0

            lax.fori_loop(0, _NBATCH, batch, 0)
            plsc.subcore_barrier()
            for off, cnt in _STRIP_PIECES:
                pltpu.sync_copy(acc.at[pl.ds(s * _STRIP + off, cnt)],
                                out_hbm.at[pl.ds(base + s * _STRIP + off, cnt)])
            plsc.subcore_barrier()


_segsum = functools.partial(
    pl.kernel,
    out_type=(jax.ShapeDtypeStruct((_NPAD, _EXT), jnp.float32),
              jax.ShapeDtypeStruct((_NPAD, _EXT), jnp.float32)),
    mesh=_MESH,
    scratch_types=[
        pltpu.VMEM_SHARED((_ACC_ROWS, _EXT), jnp.float32),
        pltpu.VMEM((_SB,), jnp.int32),
        pltpu.VMEM((_SB,), jnp.int32),
        pltpu.VMEM((1, _SB), jnp.int32),
        pltpu.VMEM((_SB, _EXT), jnp.float32),
        pltpu.SemaphoreType.DMA,
    ],
    compiler_params=pltpu.CompilerParams(use_tc_tiling_on_sc=False),
)(_segsum_body)


def _pairgather_body(p_hbm, q_hbm, ri_hbm, ci_hbm, out_hbm,
                     rowb, colb, bufp, bufq, semp, semq):
    c = lax.axis_index("c")
    s = lax.axis_index("s")
    wid = s * 2 + c
    base = wid * _EPTD

    def batch(b, _):
        off_e = base + b * 128
        pltpu.sync_copy(ri_hbm.at[pl.ds(off_e, 128)], rowb)
        pltpu.sync_copy(ci_hbm.at[pl.ds(off_e, 128)], colb)
        cp = pltpu.async_copy(p_hbm.at[rowb], bufp, semp)
        cq = pltpu.async_copy(q_hbm.at[colb], bufq, semq)
        cp.wait()
        cq.wait()

        def addrow(r, _):
            for g in range(_EXT // 16):
                sl = (r, pl.ds(g * 16, 16))
                bufp[sl] = bufp[sl] + bufq[sl]
            return 0

        lax.fori_loop(0, 128, addrow, 0)
        pltpu.sync_copy(bufp, out_hbm.at[pl.ds(off_e, 128)])
        return 0

    lax.fori_loop(0, _NBD, batch, 0)


_pairgather = functools.partial(
    pl.kernel,
    out_type=jax.ShapeDtypeStruct((_EP_PAD, _EXT), jnp.float32),
    mesh=_MESH,
    scratch_types=[
        pltpu.VMEM((128,), jnp.int32),
        pltpu.VMEM((128,), jnp.int32),
        pltpu.VMEM((128, _EXT), jnp.float32),
        pltpu.VMEM((128, _EXT), jnp.float32),
        pltpu.SemaphoreType.DMA,
        pltpu.SemaphoreType.DMA,
    ],
)(_pairgather_body)


def kernel(x, edge_index_onset, edge_index_consecutive, pot_edges, pot_chord_edges,
           batch, onsets, durations, pitches, onset_beat, duration_beat, ts_beats,
           params):
    p = params
    npad = _NPAD - _N
    xp = jnp.pad(x, ((0, npad), (0, 0)))
    feat8 = jnp.pad(
        jnp.stack([onsets, durations, pitches, onset_beat, duration_beat,
                   ts_beats, jnp.zeros_like(onsets), jnp.zeros_like(onsets)], axis=1),
        ((0, npad), (0, 0)))

    def padi(a, val):
        return jnp.pad(a, (0, _E_PAD - a.shape[0]), constant_values=val)

    sa = padi(edge_index_onset[0], 0)
    da = padi(edge_index_onset[1], _N)
    sb = padi(edge_index_consecutive[0], 0)
    db = padi(edge_index_consecutive[1], _N)
    ri = padi(pot_edges[0], 0)
    ci = padi(pot_edges[1], 0)
    zsrc = jnp.zeros((_STRIP, _EXT), jnp.float32)

    r1 = lambda a: a.reshape(1, _H)
    fWT = p["first_W"].T
    fb = r1(p["first_b"])
    s00, s01 = p["sage"][0][0], p["sage"][0][1]
    s10, s11 = p["sage"][1][0], p["sage"][1][1]
    wl0_1, wl1_1 = 0.5 * s00["Wl"].T, 0.5 * s01["Wl"].T
    wrs_1 = 0.5 * (s00["Wr"].T + s01["Wr"].T)
    bs_1 = r1(0.5 * (s00["bl"] + s01["bl"]))
    wl0_2, wl1_2 = 0.5 * s10["Wl"].T, 0.5 * s11["Wl"].T
    wrs_2 = 0.5 * (s10["Wr"].T + s11["Wr"].T)
    bs_2 = r1(0.5 * (s10["bl"] + s11["bl"]))
    sw1 = p["staff_W1"].T
    sw2 = jnp.zeros((_H, _H), jnp.float32).at[:, :2].set(p["staff_W2"].T)
    sb2 = jnp.zeros((1, _H), jnp.float32).at[0, :2].set(p["staff_b2"])
    at_ = p["dec_W1"][:, :_H].T
    bt_ = p["dec_W1"][:, _H:2 * _H].T
    cvec = jnp.stack([
        p["dec_W1"][:, 2 * _H],
        p["dec_W1"][:, 2 * _H + 1],
        p["dec_W1"][:, 2 * _H + 2],
        p["dec_b1"],
        p["dec_ln_g"],
        p["dec_ln_b"],
        p["dec_W2"][0],
        jnp.full((_H,), p["dec_b2"][0], jnp.float32),
    ], axis=0)

    h_ext = _k1(xp, fWT, fb, r1(p["first_ln_g"]), r1(p["first_ln_b"]))
    sums_on, sums_co = _segsum(h_ext, sa, da, sb, db, zsrc)
    a, stats = _k2a(sums_on, sums_co, h_ext, wl0_1, wl1_1, wrs_1, bs_1)
    h1_ext = _k2b(a, stats, r1(p["gn_w"]), r1(p["gn_b"]), r1(p["gn_ms"]))
    sums1_on, sums1_co = _segsum(h1_ext, sa, da, sb, db, zsrc)
    hidden, staffpad, p_ext, q_ext = _k3(
        sums1_on, sums1_co, h1_ext, feat8, wl0_2, wl1_2, wrs_2, bs_2,
        sw1, r1(p["staff_b1"]), r1(p["staff_ln_g"]), r1(p["staff_ln_b"]),
        sw2, sb2, at_, bt_)
    rmat = _pairgather(p_ext, q_ext, ri, ci)
    outp = _k4(rmat, cvec)
    return (outp.reshape(-1)[:_EP], staffpad[:_N, :2], hidden[:_N])


# speedup vs baseline: 2.5022x; 1.8225x over previous
import functools

import jax
import jax.numpy as jnp
from jax import lax
from jax.experimental import pallas as pl
from jax.experimental.pallas import tpu as pltpu
from jax.experimental.pallas import tpu_sc as plsc

_N = 50000
_H = 128
_EXT = 144
_E = 400000
_EP = 400000

_CHUNK = 6272
_NCHUNK_PER_CORE = 4
_NPAD = 8 * _CHUNK
_ACC_ROWS = _CHUNK + 16
_STRIP = _CHUNK // 16
_STRIP_PIECES = [(0, 128), (128, 128), (256, 128), (384, 8)]

_E_PAD = 401408
_EPT = _E_PAD // 16
_SB = 128
_NBATCH = _EPT // _SB

_EP_PAD = 401408
_EPTD = _EP_PAD // 32
_NBD = _EPTD // 128

_NB = 512
_NGRID = _NPAD // _NB
_EB = 4096
_EGRID = _EP_PAD // _EB


def _ln(v, g, b):
    m = jnp.mean(v, axis=-1, keepdims=True)
    var = jnp.mean((v - m) ** 2, axis=-1, keepdims=True)
    return (v - m) * lax.rsqrt(var + 1e-5) * g + b


def _ones_tail(nrows):
    one = jnp.ones((nrows, 1), jnp.float32)
    return jnp.concatenate([one, jnp.zeros((nrows, _EXT - _H - 1), jnp.float32)], axis=1)


def _k1_body(x_ref, w_ref, b_ref, g_ref, bb_ref, o_ref):
    v = jnp.dot(x_ref[...], w_ref[...], preferred_element_type=jnp.float32) + b_ref[...]
    v = _ln(jnp.maximum(v, 0.0), g_ref[...], bb_ref[...])
    o_ref[...] = jnp.concatenate([v, _ones_tail(v.shape[0])], axis=1)


def _k2a_body(son_ref, sco_ref, h_ref, wl0_ref, wl1_ref, wrs_ref, bs_ref,
              a_ref, st_ref):
    i = pl.program_id(0)
    mon = son_ref[:, :_H] / jnp.maximum(son_ref[:, _H:_H + 1], 1.0)
    mco = sco_ref[:, :_H] / jnp.maximum(sco_ref[:, _H:_H + 1], 1.0)
    h = h_ref[:, :_H]
    pre = (jnp.dot(mon, wl0_ref[...], preferred_element_type=jnp.float32)
           + jnp.dot(mco, wl1_ref[...], preferred_element_type=jnp.float32)
           + jnp.dot(h, wrs_ref[...], preferred_element_type=jnp.float32)
           + bs_ref[...])
    a = jnp.maximum(pre, 0.0)
    a_ref[...] = a
    rid = i * _NB + lax.broadcasted_iota(jnp.int32, (_NB, 1), 0)
    am = jnp.where(rid < _N, a, 0.0)
    contrib = jnp.concatenate(
        [jnp.sum(am, axis=0, keepdims=True),
         jnp.sum(am * am, axis=0, keepdims=True),
         jnp.zeros((6, _H), jnp.float32)], axis=0)

    @pl.when(i == 0)
    def _():
        st_ref[...] = jnp.zeros_like(st_ref)

    st_ref[...] += contrib


def _k2b_body(a_ref, st_ref, w_ref, b_ref, ms_ref, o_ref):
    a = a_ref[...]
    st = st_ref[...]
    m = st[0:1] * (1.0 / _N)
    s2 = st[1:2] * (1.0 / _N)
    ms = ms_ref[...]
    var = s2 - (2.0 * ms - ms * ms) * (m * m)
    v = (a - ms * m) * lax.rsqrt(var + 1e-5) * w_ref[...] + b_ref[...]
    o_ref[...] = jnp.concatenate([v, _ones_tail(v.shape[0])], axis=1)


def _k3_body(son_ref, sco_ref, h1_ref, ft_ref, wl0_ref, wl1_ref, wrs_ref, bs_ref,
             sw1_ref, sb1_ref, sg_ref, sbb_ref, sw2_ref, sb2_ref, at_ref, bt_ref,
             hid_ref, stf_ref, p_ref, q_ref):
    mon = son_ref[:, :_H] / jnp.maximum(son_ref[:, _H:_H + 1], 1.0)
    mco = sco_ref[:, :_H] / jnp.maximum(sco_ref[:, _H:_H + 1], 1.0)
    h = h1_ref[:, :_H]
    hid = (jnp.dot(mon, wl0_ref[...], preferred_element_type=jnp.float32)
           + jnp.dot(mco, wl1_ref[...], preferred_element_type=jnp.float32)
           + jnp.dot(h, wrs_ref[...], preferred_element_type=jnp.float32)
           + bs_ref[...])
    hid_ref[...] = hid
    t = jnp.maximum(jnp.dot(hid, sw1_ref[...], preferred_element_type=jnp.float32) + sb1_ref[...], 0.0)
    t = _ln(t, sg_ref[...], sbb_ref[...])
    stf_ref[...] = jax.nn.sigmoid(jnp.dot(t, sw2_ref[...], preferred_element_type=jnp.float32) + sb2_ref[...])
    p = jnp.dot(hid, at_ref[...], preferred_element_type=jnp.float32)
    q = jnp.dot(hid, bt_ref[...], preferred_element_type=jnp.float32)
    f = ft_ref[...]
    offb = f[:, 3:4] + f[:, 4:5]
    off = f[:, 0:1] + f[:, 1:2]
    pit = f[:, 2:3]
    z1 = jnp.zeros_like(pit)
    nz = _EXT - _H
    ptail = jnp.concatenate([offb, off, pit] + [z1] * (nz - 3), axis=1)
    qtail = jnp.concatenate([z1] * 3 + [f[:, 3:4], f[:, 5:6], f[:, 0:1], pit] + [z1] * (nz - 7), axis=1)
    p_ref[...] = jnp.concatenate([p, ptail], axis=1)
    q_ref[...] = jnp.concatenate([q, qtail], axis=1)


def _k4_body(r_ref, c_ref, o_ref):
    r = r_ref[...]
    c = c_ref[...]
    z0 = r[:, :_H]
    offb = r[:, _H]
    off = r[:, _H + 1]
    pit_r = r[:, _H + 2]
    onb = r[:, _H + 3]
    ts = r[:, _H + 4]
    ons = r[:, _H + 5]
    pit_c = r[:, _H + 6]
    os1 = 1.0 - jnp.tanh((onb - offb) / ts)
    oh = (ons == off).astype(jnp.float32)
    ps = jnp.abs(pit_c - pit_r) * (1.0 / 127.0)
    z = (z0 + os1[:, None] * c[0:1] + oh[:, None] * c[1:2]
         + ps[:, None] * c[2:3] + c[3:4])
    z = jnp.maximum(z, 0.0)
    z = _ln(z, c[4:5], c[5:6])
    out = jnp.sum(z * c[6:7], axis=-1) + c[7, 0]
    o_ref[...] = out.reshape(_EB // 128, 128)


def _node_spec(width):
    return pl.BlockSpec((_NB, width), lambda i: (i, 0))


def _full_spec(r, cdim):
    return pl.BlockSpec((r, cdim), lambda i: (0, 0))


_k1 = pl.pallas_call(
    _k1_body,
    grid=(_NGRID,),
    in_specs=[_node_spec(_H), _full_spec(_H, _H), _full_spec(1, _H),
              _full_spec(1, _H), _full_spec(1, _H)],
    out_specs=_node_spec(_EXT),
    out_shape=jax.ShapeDtypeStruct((_NPAD, _EXT), jnp.float32),
)

_k2a = pl.pallas_call(
    _k2a_body,
    grid=(_NGRID,),
    in_specs=[_node_spec(_EXT), _node_spec(_EXT), _node_spec(_EXT),
              _full_spec(_H, _H), _full_spec(_H, _H), _full_spec(_H, _H),
              _full_spec(1, _H)],
    out_specs=[_node_spec(_H), _full_spec(8, _H)],
    out_shape=[jax.ShapeDtypeStruct((_NPAD, _H), jnp.float32),
               jax.ShapeDtypeStruct((8, _H), jnp.float32)],
)

_k2b = pl.pallas_call(
    _k2b_body,
    grid=(_NGRID,),
    in_specs=[_node_spec(_H), _full_spec(8, _H), _full_spec(1, _H),
              _full_spec(1, _H), _full_spec(1, _H)],
    out_specs=_node_spec(_EXT),
    out_shape=jax.ShapeDtypeStruct((_NPAD, _EXT), jnp.float32),
)

_k3 = pl.pallas_call(
    _k3_body,
    grid=(_NGRID,),
    in_specs=[_node_spec(_EXT), _node_spec(_EXT), _node_spec(_EXT),
              _node_spec(8),
              _full_spec(_H, _H), _full_spec(_H, _H), _full_spec(_H, _H),
              _full_spec(1, _H),
              _full_spec(_H, _H), _full_spec(1, _H), _full_spec(1, _H),
              _full_spec(1, _H), _full_spec(_H, _H), _full_spec(1, _H),
              _full_spec(_H, _H), _full_spec(_H, _H)],
    out_specs=[_node_spec(_H), _node_spec(_H), _node_spec(_EXT), _node_spec(_EXT)],
    out_shape=[jax.ShapeDtypeStruct((_NPAD, _H), jnp.float32),
               jax.ShapeDtypeStruct((_NPAD, _H), jnp.float32),
               jax.ShapeDtypeStruct((_NPAD, _EXT), jnp.float32),
               jax.ShapeDtypeStruct((_NPAD, _EXT), jnp.float32)],
)

_k4 = pl.pallas_call(
    _k4_body,
    grid=(_EGRID,),
    in_specs=[pl.BlockSpec((_EB, _EXT), lambda i: (i, 0)), _full_spec(8, _H)],
    out_specs=pl.BlockSpec((_EB // 128, 128), lambda i: (i, 0)),
    out_shape=jax.ShapeDtypeStruct((_EP_PAD // 128, 128), jnp.float32),
)


_MESH = plsc.VectorSubcoreMesh(core_axis_name="c", subcore_axis_name="s")


def _segsum_body(h_hbm, sa_hbm, da_hbm, sb_hbm, db_hbm, z_hbm, outa, outb,
                 acc, srcb, dstb, lidx, rowbuf, sem, sem_s):
    c = lax.axis_index("c")
    s = lax.axis_index("s")

    for src_hbm, dst_hbm, out_hbm in ((sa_hbm, da_hbm, outa), (sb_hbm, db_hbm, outb)):
        for k in range(_NCHUNK_PER_CORE):
            base = (_NCHUNK_PER_CORE * c + k) * _CHUNK
            pltpu.sync_copy(z_hbm, acc.at[pl.ds(s * _STRIP, _STRIP)])

            @pl.when(s == 0)
            def _():
                pltpu.sync_copy(z_hbm.at[pl.ds(0, 16)], acc.at[pl.ds(_CHUNK, 16)])

            plsc.subcore_barrier()

            def stage(b, sl):
                off_e = s * _EPT + b * _SB
                pltpu.sync_copy(dst_hbm.at[pl.ds(off_e, _SB)], dstb.at[sl])
                pltpu.sync_copy(src_hbm.at[pl.ds(off_e, _SB)], srcb.at[sl])
                for g in range(_SB // 16):
                    d = dstb[sl, pl.ds(g * 16, 16)]
                    inb = (d >= base) & (d < base + _CHUNK)
                    loc = jnp.where(inb, d - base, _CHUNK)
                    lidx[sl, pl.ds(g * 16, 16)] = loc
                pltpu.async_copy(h_hbm.at[srcb.at[sl]], rowbuf.at[sl], sem)

            stage(0, 0)

            def batch(b, _):
                sl = lax.rem(b, 2)
                nsl = 1 - sl

                @pl.when(b + 1 < _NBATCH)
                def _():
                    stage(b + 1, nsl)

                pltpu.make_async_copy(
                    h_hbm.at[srcb.at[sl]], rowbuf.at[sl], sem).wait()
                pltpu.async_copy(
                    rowbuf.at[sl], acc.at[lidx.at[sl]], sem_s, add=True).wait()
                return 0

            lax.fori_loop(0, _NBATCH, batch, 0)
            plsc.subcore_barrier()
            for off, cnt in _STRIP_PIECES:
                pltpu.sync_copy(acc.at[pl.ds(s * _STRIP + off, cnt)],
                                out_hbm.at[pl.ds(base + s * _STRIP + off, cnt)])
            plsc.subcore_barrier()


_segsum = functools.partial(
    pl.kernel,
    out_type=(jax.ShapeDtypeStruct((_NPAD, _EXT), jnp.float32),
              jax.ShapeDtypeStruct((_NPAD, _EXT), jnp.float32)),
    mesh=_MESH,
    scratch_types=[
        pltpu.VMEM_SHARED((_ACC_ROWS, _EXT), jnp.float32),
        pltpu.VMEM((2, _SB), jnp.int32),
        pltpu.VMEM((2, _SB), jnp.int32),
        pltpu.VMEM((2, _SB), jnp.int32),
        pltpu.VMEM((2, _SB, _EXT), jnp.float32),
        pltpu.SemaphoreType.DMA,
        pltpu.SemaphoreType.DMA,
    ],
    compiler_params=pltpu.CompilerParams(use_tc_tiling_on_sc=False),
)(_segsum_body)


def _pairgather_body(p_hbm, q_hbm, ri_hbm, ci_hbm, out_hbm,
                     rowb, colb, bufp, bufq, semp, semq):
    c = lax.axis_index("c")
    s = lax.axis_index("s")
    wid = s * 2 + c
    base = wid * _EPTD

    def stage(b, sl):
        off_e = base + b * 128
        pltpu.sync_copy(ri_hbm.at[pl.ds(off_e, 128)], rowb.at[sl])
        pltpu.sync_copy(ci_hbm.at[pl.ds(off_e, 128)], colb.at[sl])
        pltpu.async_copy(p_hbm.at[rowb.at[sl]], bufp.at[sl], semp)
        pltpu.async_copy(q_hbm.at[colb.at[sl]], bufq.at[sl], semq)

    stage(0, 0)

    def batch(b, _):
        sl = lax.rem(b, 2)
        nsl = 1 - sl

        @pl.when(b + 1 < _NBD)
        def _():
            stage(b + 1, nsl)

        pltpu.make_async_copy(p_hbm.at[rowb.at[sl]], bufp.at[sl], semp).wait()
        pltpu.make_async_copy(q_hbm.at[colb.at[sl]], bufq.at[sl], semq).wait()

        def addrow(r, _):
            for g in range(_EXT // 16):
                bufp[sl, r, pl.ds(g * 16, 16)] = (
                    bufp[sl, r, pl.ds(g * 16, 16)] + bufq[sl, r, pl.ds(g * 16, 16)])
            return 0

        lax.fori_loop(0, 128, addrow, 0)
        pltpu.sync_copy(bufp.at[sl], out_hbm.at[pl.ds(base + b * 128, 128)])
        return 0

    lax.fori_loop(0, _NBD, batch, 0)


_pairgather = functools.partial(
    pl.kernel,
    out_type=jax.ShapeDtypeStruct((_EP_PAD, _EXT), jnp.float32),
    mesh=_MESH,
    scratch_types=[
        pltpu.VMEM((2, 128), jnp.int32),
        pltpu.VMEM((2, 128), jnp.int32),
        pltpu.VMEM((2, 128, _EXT), jnp.float32),
        pltpu.VMEM((2, 128, _EXT), jnp.float32),
        pltpu.SemaphoreType.DMA,
        pltpu.SemaphoreType.DMA,
    ],
    compiler_params=pltpu.CompilerParams(use_tc_tiling_on_sc=False),
)(_pairgather_body)


def kernel(x, edge_index_onset, edge_index_consecutive, pot_edges, pot_chord_edges,
           batch, onsets, durations, pitches, onset_beat, duration_beat, ts_beats,
           params):
    p = params
    npad = _NPAD - _N
    xp = jnp.pad(x, ((0, npad), (0, 0)))
    feat8 = jnp.pad(
        jnp.stack([onsets, durations, pitches, onset_beat, duration_beat,
                   ts_beats, jnp.zeros_like(onsets), jnp.zeros_like(onsets)], axis=1),
        ((0, npad), (0, 0)))

    def padi(a, val):
        return jnp.pad(a, (0, _E_PAD - a.shape[0]), constant_values=val)

    sa = padi(edge_index_onset[0], 0)
    da = padi(edge_index_onset[1], _N)
    sb = padi(edge_index_consecutive[0], 0)
    db = padi(edge_index_consecutive[1], _N)
    ri = padi(pot_edges[0], 0)
    ci = padi(pot_edges[1], 0)
    zsrc = jnp.zeros((_STRIP, _EXT), jnp.float32)

    r1 = lambda a: a.reshape(1, _H)
    fWT = p["first_W"].T
    fb = r1(p["first_b"])
    s00, s01 = p["sage"][0][0], p["sage"][0][1]
    s10, s11 = p["sage"][1][0], p["sage"][1][1]
    wl0_1, wl1_1 = 0.5 * s00["Wl"].T, 0.5 * s01["Wl"].T
    wrs_1 = 0.5 * (s00["Wr"].T + s01["Wr"].T)
    bs_1 = r1(0.5 * (s00["bl"] + s01["bl"]))
    wl0_2, wl1_2 = 0.5 * s10["Wl"].T, 0.5 * s11["Wl"].T
    wrs_2 = 0.5 * (s10["Wr"].T + s11["Wr"].T)
    bs_2 = r1(0.5 * (s10["bl"] + s11["bl"]))
    sw1 = p["staff_W1"].T
    sw2 = jnp.zeros((_H, _H), jnp.float32).at[:, :2].set(p["staff_W2"].T)
    sb2 = jnp.zeros((1, _H), jnp.float32).at[0, :2].set(p["staff_b2"])
    at_ = p["dec_W1"][:, :_H].T
    bt_ = p["dec_W1"][:, _H:2 * _H].T
    cvec = jnp.stack([
        p["dec_W1"][:, 2 * _H],
        p["dec_W1"][:, 2 * _H + 1],
        p["dec_W1"][:, 2 * _H + 2],
        p["dec_b1"],
        p["dec_ln_g"],
        p["dec_ln_b"],
        p["dec_W2"][0],
        jnp.full((_H,), p["dec_b2"][0], jnp.float32),
    ], axis=0)

    h_ext = _k1(xp, fWT, fb, r1(p["first_ln_g"]), r1(p["first_ln_b"]))
    sums_on, sums_co = _segsum(h_ext, sa, da, sb, db, zsrc)
    a, stats = _k2a(sums_on, sums_co, h_ext, wl0_1, wl1_1, wrs_1, bs_1)
    h1_ext = _k2b(a, stats, r1(p["gn_w"]), r1(p["gn_b"]), r1(p["gn_ms"]))
    sums1_on, sums1_co = _segsum(h1_ext, sa, da, sb, db, zsrc)
    hidden, staffpad, p_ext, q_ext = _k3(
        sums1_on, sums1_co, h1_ext, feat8, wl0_2, wl1_2, wrs_2, bs_2,
        sw1, r1(p["staff_b1"]), r1(p["staff_ln_g"]), r1(p["staff_ln_b"]),
        sw2, sb2, at_, bt_)
    rmat = _pairgather(p_ext, q_ext, ri, ci)
    outp = _k4(rmat, cvec)
    return (outp.reshape(-1)[:_EP], staffpad[:_N, :2], hidden[:_N])


# speedup vs baseline: 5.0594x; 2.0220x over previous
import functools

import jax
import jax.numpy as jnp
from jax import lax
from jax.experimental import pallas as pl
from jax.experimental.pallas import tpu as pltpu
from jax.experimental.pallas import tpu_sc as plsc

_N = 50000
_H = 128
_EXT = 144
_E = 400000
_EP = 400000

_CHUNK = 6272
_NCHUNK_PER_CORE = 4
_NPAD = 8 * _CHUNK
_ACC_ROWS = _CHUNK + 16
_STRIP = _CHUNK // 16
_STRIP_PIECES = [(0, 128), (128, 128), (256, 128), (384, 8)]

_E_PAD = 401408
_EPT = _E_PAD // 16
_SB = 128
_NBATCH = _EPT // _SB

_EP_PAD = 401408
_EPTD = _EP_PAD // 32
_NBD = _EPTD // 128

_NB = 512
_NGRID = _NPAD // _NB
_EB = 4096
_EGRID = _EP_PAD // _EB


def _ln(v, g, b):
    m = jnp.mean(v, axis=-1, keepdims=True)
    var = jnp.mean((v - m) ** 2, axis=-1, keepdims=True)
    return (v - m) * lax.rsqrt(var + 1e-5) * g + b


def _ones_tail(nrows):
    one = jnp.ones((nrows, 1), jnp.float32)
    return jnp.concatenate([one, jnp.zeros((nrows, _EXT - _H - 1), jnp.float32)], axis=1)


def _k1_body(x_ref, w_ref, b_ref, g_ref, bb_ref, o_ref):
    v = jnp.dot(x_ref[...], w_ref[...], preferred_element_type=jnp.float32) + b_ref[...]
    v = _ln(jnp.maximum(v, 0.0), g_ref[...], bb_ref[...])
    o_ref[...] = jnp.concatenate([v, _ones_tail(v.shape[0])], axis=1)


def _k2a_body(son_ref, sco_ref, h_ref, wl0_ref, wl1_ref, wrs_ref, bs_ref,
              a_ref, st_ref):
    i = pl.program_id(0)
    mon = son_ref[:, :_H] / jnp.maximum(son_ref[:, _H:_H + 1], 1.0)
    mco = sco_ref[:, :_H] / jnp.maximum(sco_ref[:, _H:_H + 1], 1.0)
    h = h_ref[:, :_H]
    pre = (jnp.dot(mon, wl0_ref[...], preferred_element_type=jnp.float32)
           + jnp.dot(mco, wl1_ref[...], preferred_element_type=jnp.float32)
           + jnp.dot(h, wrs_ref[...], preferred_element_type=jnp.float32)
           + bs_ref[...])
    a = jnp.maximum(pre, 0.0)
    a_ref[...] = a
    rid = i * _NB + lax.broadcasted_iota(jnp.int32, (_NB, 1), 0)
    am = jnp.where(rid < _N, a, 0.0)
    contrib = jnp.concatenate(
        [jnp.sum(am, axis=0, keepdims=True),
         jnp.sum(am * am, axis=0, keepdims=True),
         jnp.zeros((6, _H), jnp.float32)], axis=0)

    @pl.when(i == 0)
    def _():
        st_ref[...] = jnp.zeros_like(st_ref)

    st_ref[...] += contrib


def _k2b_body(a_ref, st_ref, w_ref, b_ref, ms_ref, o_ref):
    a = a_ref[...]
    st = st_ref[...]
    m = st[0:1] * (1.0 / _N)
    s2 = st[1:2] * (1.0 / _N)
    ms = ms_ref[...]
    var = s2 - (2.0 * ms - ms * ms) * (m * m)
    v = (a - ms * m) * lax.rsqrt(var + 1e-5) * w_ref[...] + b_ref[...]
    o_ref[...] = jnp.concatenate([v, _ones_tail(v.shape[0])], axis=1)


def _k3_body(son_ref, sco_ref, h1_ref, ft_ref, wl0_ref, wl1_ref, wrs_ref, bs_ref,
             sw1_ref, sb1_ref, sg_ref, sbb_ref, sw2_ref, sb2_ref, at_ref, bt_ref,
             hid_ref, stf_ref, p_ref, q_ref):
    mon = son_ref[:, :_H] / jnp.maximum(son_ref[:, _H:_H + 1], 1.0)
    mco = sco_ref[:, :_H] / jnp.maximum(sco_ref[:, _H:_H + 1], 1.0)
    h = h1_ref[:, :_H]
    hid = (jnp.dot(mon, wl0_ref[...], preferred_element_type=jnp.float32)
           + jnp.dot(mco, wl1_ref[...], preferred_element_type=jnp.float32)
           + jnp.dot(h, wrs_ref[...], preferred_element_type=jnp.float32)
           + bs_ref[...])
    hid_ref[...] = hid
    t = jnp.maximum(jnp.dot(hid, sw1_ref[...], preferred_element_type=jnp.float32) + sb1_ref[...], 0.0)
    t = _ln(t, sg_ref[...], sbb_ref[...])
    stf_ref[...] = jax.nn.sigmoid(jnp.dot(t, sw2_ref[...], preferred_element_type=jnp.float32) + sb2_ref[...])
    p = jnp.dot(hid, at_ref[...], preferred_element_type=jnp.float32)
    q = jnp.dot(hid, bt_ref[...], preferred_element_type=jnp.float32)
    f = ft_ref[...]
    offb = f[:, 3:4] + f[:, 4:5]
    off = f[:, 0:1] + f[:, 1:2]
    pit = f[:, 2:3]
    z1 = jnp.zeros_like(pit)
    nz = _EXT - _H
    ptail = jnp.concatenate([offb, off, pit] + [z1] * (nz - 3), axis=1)
    qtail = jnp.concatenate([z1] * 3 + [f[:, 3:4], f[:, 5:6], f[:, 0:1], pit] + [z1] * (nz - 7), axis=1)
    p_ref[...] = jnp.concatenate([p, ptail], axis=1)
    q_ref[...] = jnp.concatenate([q, qtail], axis=1)


def _k4_body(r_ref, c_ref, o_ref):
    r = r_ref[...]
    c = c_ref[...]
    z0 = r[:, :_H]
    offb = r[:, _H]
    off = r[:, _H + 1]
    pit_r = r[:, _H + 2]
    onb = r[:, _H + 3]
    ts = r[:, _H + 4]
    ons = r[:, _H + 5]
    pit_c = r[:, _H + 6]
    os1 = 1.0 - jnp.tanh((onb - offb) / ts)
    oh = (ons == off).astype(jnp.float32)
    ps = jnp.abs(pit_c - pit_r) * (1.0 / 127.0)
    z = (z0 + os1[:, None] * c[0:1] + oh[:, None] * c[1:2]
         + ps[:, None] * c[2:3] + c[3:4])
    z = jnp.maximum(z, 0.0)
    z = _ln(z, c[4:5], c[5:6])
    out = jnp.sum(z * c[6:7], axis=-1) + c[7, 0]
    o_ref[...] = out.reshape(_EB // 128, 128)


def _node_spec(width):
    return pl.BlockSpec((_NB, width), lambda i: (i, 0))


def _full_spec(r, cdim):
    return pl.BlockSpec((r, cdim), lambda i: (0, 0))


_k1 = pl.pallas_call(
    _k1_body,
    grid=(_NGRID,),
    in_specs=[_node_spec(_H), _full_spec(_H, _H), _full_spec(1, _H),
              _full_spec(1, _H), _full_spec(1, _H)],
    out_specs=_node_spec(_EXT),
    out_shape=jax.ShapeDtypeStruct((_NPAD, _EXT), jnp.float32),
)

_k2a = pl.pallas_call(
    _k2a_body,
    grid=(_NGRID,),
    in_specs=[_node_spec(_EXT), _node_spec(_EXT), _node_spec(_EXT),
              _full_spec(_H, _H), _full_spec(_H, _H), _full_spec(_H, _H),
              _full_spec(1, _H)],
    out_specs=[_node_spec(_H), _full_spec(8, _H)],
    out_shape=[jax.ShapeDtypeStruct((_NPAD, _H), jnp.float32),
               jax.ShapeDtypeStruct((8, _H), jnp.float32)],
)

_k2b = pl.pallas_call(
    _k2b_body,
    grid=(_NGRID,),
    in_specs=[_node_spec(_H), _full_spec(8, _H), _full_spec(1, _H),
              _full_spec(1, _H), _full_spec(1, _H)],
    out_specs=_node_spec(_EXT),
    out_shape=jax.ShapeDtypeStruct((_NPAD, _EXT), jnp.float32),
)

_k3 = pl.pallas_call(
    _k3_body,
    grid=(_NGRID,),
    in_specs=[_node_spec(_EXT), _node_spec(_EXT), _node_spec(_EXT),
              _node_spec(8),
              _full_spec(_H, _H), _full_spec(_H, _H), _full_spec(_H, _H),
              _full_spec(1, _H),
              _full_spec(_H, _H), _full_spec(1, _H), _full_spec(1, _H),
              _full_spec(1, _H), _full_spec(_H, _H), _full_spec(1, _H),
              _full_spec(_H, _H), _full_spec(_H, _H)],
    out_specs=[_node_spec(_H), _node_spec(_H), _node_spec(_EXT), _node_spec(_EXT)],
    out_shape=[jax.ShapeDtypeStruct((_NPAD, _H), jnp.float32),
               jax.ShapeDtypeStruct((_NPAD, _H), jnp.float32),
               jax.ShapeDtypeStruct((_NPAD, _EXT), jnp.float32),
               jax.ShapeDtypeStruct((_NPAD, _EXT), jnp.float32)],
)

_k4 = pl.pallas_call(
    _k4_body,
    grid=(_EGRID,),
    in_specs=[pl.BlockSpec((_EB, _EXT), lambda i: (i, 0)), _full_spec(8, _H)],
    out_specs=pl.BlockSpec((_EB // 128, 128), lambda i: (i, 0)),
    out_shape=jax.ShapeDtypeStruct((_EP_PAD // 128, 128), jnp.float32),
)


_MESH = plsc.VectorSubcoreMesh(core_axis_name="c", subcore_axis_name="s")


_SEG = 1792
_NSEG = _EPT // _SEG
_CCAP = 2304


def _segsum_body(h_hbm, sa_hbm, da_hbm, sb_hbm, db_hbm, z_hbm, outa, outb,
                 acc, sseg, dseg, csrc, cloc, rowbuf, sem, sem_s):
    c = lax.axis_index("c")
    s = lax.axis_index("s")
    zeros16 = jnp.zeros((16,), jnp.int32)
    dump16 = jnp.full((16,), _CHUNK, jnp.int32)

    def drain(nb):
        def dbody(i, _):
            pltpu.async_copy(
                h_hbm.at[csrc.at[pl.ds(i * 128, 128)]], rowbuf, sem).wait()
            pltpu.async_copy(
                rowbuf, acc.at[cloc.at[pl.ds(i * 128, 128)]], sem_s, add=True).wait()
            return 0
        lax.fori_loop(0, nb, dbody, 0)

    for src_hbm, dst_hbm, out_hbm in ((sa_hbm, da_hbm, outa), (sb_hbm, db_hbm, outb)):
        for k in range(_NCHUNK_PER_CORE):
            base = (_NCHUNK_PER_CORE * c + k) * _CHUNK
            pltpu.sync_copy(z_hbm, acc.at[pl.ds(s * _STRIP, _STRIP)])

            @pl.when(s == 0)
            def _():
                pltpu.sync_copy(z_hbm.at[pl.ds(0, 16)], acc.at[pl.ds(_CHUNK, 16)])

            plsc.subcore_barrier()

            def seg_body(g2, cofs):
                off_e = s * _EPT + g2 * _SEG
                pltpu.sync_copy(dst_hbm.at[pl.ds(off_e, _SEG)], dseg)
                pltpu.sync_copy(src_hbm.at[pl.ds(off_e, _SEG)], sseg)

                def grp(g, ofs):
                    d = dseg[pl.ds(g * 16, 16)]
                    sv = sseg[pl.ds(g * 16, 16)]
                    inb = (d >= base) & (d < base + _CHUNK)
                    m32 = jnp.where(inb, jnp.int32(1), jnp.int32(0))
                    pos = ofs + plsc.cumsum(m32) - 1
                    plsc.store_scatter(csrc, [pos], sv, mask=inb)
                    plsc.store_scatter(cloc, [pos], d - base, mask=inb)
                    return ofs + jnp.sum(m32)

                cofs = lax.fori_loop(0, _SEG // 16, grp, cofs)
                nfull = lax.div(cofs, 128)
                drain(nfull)
                rem = cofs - nfull * 128

                def mv(j, _):
                    vs = csrc[pl.ds(nfull * 128 + j * 16, 16)]
                    vl = cloc[pl.ds(nfull * 128 + j * 16, 16)]
                    csrc[pl.ds(j * 16, 16)] = vs
                    cloc[pl.ds(j * 16, 16)] = vl
                    return 0

                lax.fori_loop(0, lax.div(rem + 15, 16), mv, 0)
                return rem

            cofs = lax.fori_loop(0, _NSEG, seg_body, jnp.int32(0))

            def pad(j, _):
                csrc[pl.ds(cofs + j * 16, 16)] = zeros16
                cloc[pl.ds(cofs + j * 16, 16)] = dump16
                return 0

            lax.fori_loop(0, 8, pad, 0)
            drain(lax.div(cofs + 127, 128))
            plsc.subcore_barrier()
            for off, cnt in _STRIP_PIECES:
                pltpu.sync_copy(acc.at[pl.ds(s * _STRIP + off, cnt)],
                                out_hbm.at[pl.ds(base + s * _STRIP + off, cnt)])
            plsc.subcore_barrier()


_segsum = functools.partial(
    pl.kernel,
    out_type=(jax.ShapeDtypeStruct((_NPAD, _EXT), jnp.float32),
              jax.ShapeDtypeStruct((_NPAD, _EXT), jnp.float32)),
    mesh=_MESH,
    scratch_types=[
        pltpu.VMEM_SHARED((_ACC_ROWS, _EXT), jnp.float32),
        pltpu.VMEM((_SEG,), jnp.int32),
        pltpu.VMEM((_SEG,), jnp.int32),
        pltpu.VMEM((_CCAP,), jnp.int32),
        pltpu.VMEM((_CCAP,), jnp.int32),
        pltpu.VMEM((_SB, _EXT), jnp.float32),
        pltpu.SemaphoreType.DMA,
        pltpu.SemaphoreType.DMA,
    ],
    compiler_params=pltpu.CompilerParams(use_tc_tiling_on_sc=False,
                                         needs_layout_passes=False),
)(_segsum_body)


def _pairgather_body(p_hbm, q_hbm, ri_hbm, ci_hbm, out_hbm,
                     rowb, colb, bufp, bufq, semp, semq):
    c = lax.axis_index("c")
    s = lax.axis_index("s")
    wid = s * 2 + c
    base = wid * _EPTD

    def stage(b, sl):
        off_e = base + b * 128
        pltpu.sync_copy(ri_hbm.at[pl.ds(off_e, 128)], rowb.at[sl])
        pltpu.sync_copy(ci_hbm.at[pl.ds(off_e, 128)], colb.at[sl])
        pltpu.async_copy(p_hbm.at[rowb.at[sl]], bufp.at[sl], semp)
        pltpu.async_copy(q_hbm.at[colb.at[sl]], bufq.at[sl], semq)

    stage(0, 0)

    def batch(b, _):
        sl = lax.rem(b, 2)
        nsl = 1 - sl

        @pl.when(b + 1 < _NBD)
        def _():
            stage(b + 1, nsl)

        pltpu.make_async_copy(p_hbm.at[rowb.at[sl]], bufp.at[sl], semp).wait()
        pltpu.make_async_copy(q_hbm.at[colb.at[sl]], bufq.at[sl], semq).wait()

        def addrow(r, _):
            for g in range(_EXT // 16):
                bufp[sl, r, pl.ds(g * 16, 16)] = (
                    bufp[sl, r, pl.ds(g * 16, 16)] + bufq[sl, r, pl.ds(g * 16, 16)])
            return 0

        lax.fori_loop(0, 128, addrow, 0)
        pltpu.sync_copy(bufp.at[sl], out_hbm.at[pl.ds(base + b * 128, 128)])
        return 0

    lax.fori_loop(0, _NBD, batch, 0)


_pairgather = functools.partial(
    pl.kernel,
    out_type=jax.ShapeDtypeStruct((_EP_PAD, _EXT), jnp.float32),
    mesh=_MESH,
    scratch_types=[
        pltpu.VMEM((2, 128), jnp.int32),
        pltpu.VMEM((2, 128), jnp.int32),
        pltpu.VMEM((2, 128, _EXT), jnp.float32),
        pltpu.VMEM((2, 128, _EXT), jnp.float32),
        pltpu.SemaphoreType.DMA,
        pltpu.SemaphoreType.DMA,
    ],
    compiler_params=pltpu.CompilerParams(use_tc_tiling_on_sc=False),
)(_pairgather_body)


def kernel(x, edge_index_onset, edge_index_consecutive, pot_edges, pot_chord_edges,
           batch, onsets, durations, pitches, onset_beat, duration_beat, ts_beats,
           params):
    p = params
    npad = _NPAD - _N
    xp = jnp.pad(x, ((0, npad), (0, 0)))
    feat8 = jnp.pad(
        jnp.stack([onsets, durations, pitches, onset_beat, duration_beat,
                   ts_beats, jnp.zeros_like(onsets), jnp.zeros_like(onsets)], axis=1),
        ((0, npad), (0, 0)))

    def padi(a, val):
        return jnp.pad(a, (0, _E_PAD - a.shape[0]), constant_values=val)

    sa = padi(edge_index_onset[0], 0)
    da = padi(edge_index_onset[1], _N)
    sb = padi(edge_index_consecutive[0], 0)
    db = padi(edge_index_consecutive[1], _N)
    ri = padi(pot_edges[0], 0)
    ci = padi(pot_edges[1], 0)
    zsrc = jnp.zeros((_STRIP, _EXT), jnp.float32)

    r1 = lambda a: a.reshape(1, _H)
    fWT = p["first_W"].T
    fb = r1(p["first_b"])
    s00, s01 = p["sage"][0][0], p["sage"][0][1]
    s10, s11 = p["sage"][1][0], p["sage"][1][1]
    wl0_1, wl1_1 = 0.5 * s00["Wl"].T, 0.5 * s01["Wl"].T
    wrs_1 = 0.5 * (s00["Wr"].T + s01["Wr"].T)
    bs_1 = r1(0.5 * (s00["bl"] + s01["bl"]))
    wl0_2, wl1_2 = 0.5 * s10["Wl"].T, 0.5 * s11["Wl"].T
    wrs_2 = 0.5 * (s10["Wr"].T + s11["Wr"].T)
    bs_2 = r1(0.5 * (s10["bl"] + s11["bl"]))
    sw1 = p["staff_W1"].T
    sw2 = jnp.zeros((_H, _H), jnp.float32).at[:, :2].set(p["staff_W2"].T)
    sb2 = jnp.zeros((1, _H), jnp.float32).at[0, :2].set(p["staff_b2"])
    at_ = p["dec_W1"][:, :_H].T
    bt_ = p["dec_W1"][:, _H:2 * _H].T
    cvec = jnp.stack([
        p["dec_W1"][:, 2 * _H],
        p["dec_W1"][:, 2 * _H + 1],
        p["dec_W1"][:, 2 * _H + 2],
        p["dec_b1"],
        p["dec_ln_g"],
        p["dec_ln_b"],
        p["dec_W2"][0],
        jnp.full((_H,), p["dec_b2"][0], jnp.float32),
    ], axis=0)

    h_ext = _k1(xp, fWT, fb, r1(p["first_ln_g"]), r1(p["first_ln_b"]))
    sums_on, sums_co = _segsum(h_ext, sa, da, sb, db, zsrc)
    a, stats = _k2a(sums_on, sums_co, h_ext, wl0_1, wl1_1, wrs_1, bs_1)
    h1_ext = _k2b(a, stats, r1(p["gn_w"]), r1(p["gn_b"]), r1(p["gn_ms"]))
    sums1_on, sums1_co = _segsum(h1_ext, sa, da, sb, db, zsrc)
    hidden, staffpad, p_ext, q_ext = _k3(
        sums1_on, sums1_co, h1_ext, feat8, wl0_2, wl1_2, wrs_2, bs_2,
        sw1, r1(p["staff_b1"]), r1(p["staff_ln_g"]), r1(p["staff_ln_b"]),
        sw2, sb2, at_, bt_)
    rmat = _pairgather(p_ext, q_ext, ri, ci)
    outp = _k4(rmat, cvec)
    return (outp.reshape(-1)[:_EP], staffpad[:_N, :2], hidden[:_N])


# speedup vs baseline: 5.2680x; 1.0412x over previous
import functools

import jax
import jax.numpy as jnp
from jax import lax
from jax.experimental import pallas as pl
from jax.experimental.pallas import tpu as pltpu
from jax.experimental.pallas import tpu_sc as plsc

_N = 50000
_H = 128
_EXT = 144
_E = 400000
_EP = 400000

_CHUNK = 6272
_NCHUNK_PER_CORE = 4
_NPAD = 8 * _CHUNK
_ACC_ROWS = _CHUNK + 16
_STRIP = _CHUNK // 16
_STRIP_PIECES = [(0, 128), (128, 128), (256, 128), (384, 8)]

_E_PAD = 401408
_EPT = _E_PAD // 16
_SB = 128
_NBATCH = _EPT // _SB

_EP_PAD = 401408
_EPTD = _EP_PAD // 32
_NBD = _EPTD // 128

_NB = 512
_NGRID = _NPAD // _NB
_EB = 4096
_EGRID = _EP_PAD // _EB


def _ln(v, g, b):
    m = jnp.mean(v, axis=-1, keepdims=True)
    var = jnp.mean((v - m) ** 2, axis=-1, keepdims=True)
    return (v - m) * lax.rsqrt(var + 1e-5) * g + b


def _ones_tail(nrows):
    one = jnp.ones((nrows, 1), jnp.float32)
    return jnp.concatenate([one, jnp.zeros((nrows, _EXT - _H - 1), jnp.float32)], axis=1)


def _k1_body(x_ref, w_ref, b_ref, g_ref, bb_ref, o_ref):
    v = jnp.dot(x_ref[...], w_ref[...], preferred_element_type=jnp.float32) + b_ref[...]
    v = _ln(jnp.maximum(v, 0.0), g_ref[...], bb_ref[...])
    o_ref[...] = jnp.concatenate([v, _ones_tail(v.shape[0])], axis=1)


def _k2a_body(son_ref, sco_ref, h_ref, wl0_ref, wl1_ref, wrs_ref, bs_ref,
              a_ref, st_ref):
    i = pl.program_id(0)
    mon = son_ref[:, :_H] / jnp.maximum(son_ref[:, _H:_H + 1], 1.0)
    mco = sco_ref[:, :_H] / jnp.maximum(sco_ref[:, _H:_H + 1], 1.0)
    h = h_ref[:, :_H]
    pre = (jnp.dot(mon, wl0_ref[...], preferred_element_type=jnp.float32)
           + jnp.dot(mco, wl1_ref[...], preferred_element_type=jnp.float32)
           + jnp.dot(h, wrs_ref[...], preferred_element_type=jnp.float32)
           + bs_ref[...])
    a = jnp.maximum(pre, 0.0)
    a_ref[...] = a
    rid = i * _NB + lax.broadcasted_iota(jnp.int32, (_NB, 1), 0)
    am = jnp.where(rid < _N, a, 0.0)
    contrib = jnp.concatenate(
        [jnp.sum(am, axis=0, keepdims=True),
         jnp.sum(am * am, axis=0, keepdims=True),
         jnp.zeros((6, _H), jnp.float32)], axis=0)

    @pl.when(i == 0)
    def _():
        st_ref[...] = jnp.zeros_like(st_ref)

    st_ref[...] += contrib


def _k2b_body(a_ref, st_ref, w_ref, b_ref, ms_ref, o_ref):
    a = a_ref[...]
    st = st_ref[...]
    m = st[0:1] * (1.0 / _N)
    s2 = st[1:2] * (1.0 / _N)
    ms = ms_ref[...]
    var = s2 - (2.0 * ms - ms * ms) * (m * m)
    v = (a - ms * m) * lax.rsqrt(var + 1e-5) * w_ref[...] + b_ref[...]
    o_ref[...] = jnp.concatenate([v, _ones_tail(v.shape[0])], axis=1)


def _k3_body(son_ref, sco_ref, h1_ref, ft_ref, wl0_ref, wl1_ref, wrs_ref, bs_ref,
             sw1_ref, sb1_ref, sg_ref, sbb_ref, sw2_ref, sb2_ref, at_ref, bt_ref,
             hid_ref, stf_ref, p_ref, q_ref):
    mon = son_ref[:, :_H] / jnp.maximum(son_ref[:, _H:_H + 1], 1.0)
    mco = sco_ref[:, :_H] / jnp.maximum(sco_ref[:, _H:_H + 1], 1.0)
    h = h1_ref[:, :_H]
    hid = (jnp.dot(mon, wl0_ref[...], preferred_element_type=jnp.float32)
           + jnp.dot(mco, wl1_ref[...], preferred_element_type=jnp.float32)
           + jnp.dot(h, wrs_ref[...], preferred_element_type=jnp.float32)
           + bs_ref[...])
    hid_ref[...] = hid
    t = jnp.maximum(jnp.dot(hid, sw1_ref[...], preferred_element_type=jnp.float32) + sb1_ref[...], 0.0)
    t = _ln(t, sg_ref[...], sbb_ref[...])
    stf_ref[...] = jax.nn.sigmoid(jnp.dot(t, sw2_ref[...], preferred_element_type=jnp.float32) + sb2_ref[...])
    p = jnp.dot(hid, at_ref[...], preferred_element_type=jnp.float32)
    q = jnp.dot(hid, bt_ref[...], preferred_element_type=jnp.float32)
    f = ft_ref[...]
    offb = f[:, 3:4] + f[:, 4:5]
    off = f[:, 0:1] + f[:, 1:2]
    pit = f[:, 2:3]
    z1 = jnp.zeros_like(pit)
    nz = _EXT - _H
    ptail = jnp.concatenate([offb, off, pit] + [z1] * (nz - 3), axis=1)
    qtail = jnp.concatenate([z1] * 3 + [f[:, 3:4], f[:, 5:6], f[:, 0:1], pit] + [z1] * (nz - 7), axis=1)
    p_ref[...] = jnp.concatenate([p, ptail], axis=1)
    q_ref[...] = jnp.concatenate([q, qtail], axis=1)


def _k4_body(r_ref, c_ref, o_ref):
    r = r_ref[...]
    c = c_ref[...]
    z0 = r[:, :_H]
    offb = r[:, _H]
    off = r[:, _H + 1]
    pit_r = r[:, _H + 2]
    onb = r[:, _H + 3]
    ts = r[:, _H + 4]
    ons = r[:, _H + 5]
    pit_c = r[:, _H + 6]
    os1 = 1.0 - jnp.tanh((onb - offb) / ts)
    oh = (ons == off).astype(jnp.float32)
    ps = jnp.abs(pit_c - pit_r) * (1.0 / 127.0)
    z = (z0 + os1[:, None] * c[0:1] + oh[:, None] * c[1:2]
         + ps[:, None] * c[2:3] + c[3:4])
    z = jnp.maximum(z, 0.0)
    z = _ln(z, c[4:5], c[5:6])
    out = jnp.sum(z * c[6:7], axis=-1) + c[7, 0]
    o_ref[...] = out.reshape(_EB // 128, 128)


def _node_spec(width):
    return pl.BlockSpec((_NB, width), lambda i: (i, 0))


def _full_spec(r, cdim):
    return pl.BlockSpec((r, cdim), lambda i: (0, 0))


_k1 = pl.pallas_call(
    _k1_body,
    grid=(_NGRID,),
    in_specs=[_node_spec(_H), _full_spec(_H, _H), _full_spec(1, _H),
              _full_spec(1, _H), _full_spec(1, _H)],
    out_specs=_node_spec(_EXT),
    out_shape=jax.ShapeDtypeStruct((_NPAD, _EXT), jnp.float32),
)

_k2a = pl.pallas_call(
    _k2a_body,
    grid=(_NGRID,),
    in_specs=[_node_spec(_EXT), _node_spec(_EXT), _node_spec(_EXT),
              _full_spec(_H, _H), _full_spec(_H, _H), _full_spec(_H, _H),
              _full_spec(1, _H)],
    out_specs=[_node_spec(_H), _full_spec(8, _H)],
    out_shape=[jax.ShapeDtypeStruct((_NPAD, _H), jnp.float32),
               jax.ShapeDtypeStruct((8, _H), jnp.float32)],
)

_k2b = pl.pallas_call(
    _k2b_body,
    grid=(_NGRID,),
    in_specs=[_node_spec(_H), _full_spec(8, _H), _full_spec(1, _H),
              _full_spec(1, _H), _full_spec(1, _H)],
    out_specs=_node_spec(_EXT),
    out_shape=jax.ShapeDtypeStruct((_NPAD, _EXT), jnp.float32),
)

_k3 = pl.pallas_call(
    _k3_body,
    grid=(_NGRID,),
    in_specs=[_node_spec(_EXT), _node_spec(_EXT), _node_spec(_EXT),
              _node_spec(8),
              _full_spec(_H, _H), _full_spec(_H, _H), _full_spec(_H, _H),
              _full_spec(1, _H),
              _full_spec(_H, _H), _full_spec(1, _H), _full_spec(1, _H),
              _full_spec(1, _H), _full_spec(_H, _H), _full_spec(1, _H),
              _full_spec(_H, _H), _full_spec(_H, _H)],
    out_specs=[_node_spec(_H), _node_spec(_H), _node_spec(_EXT), _node_spec(_EXT)],
    out_shape=[jax.ShapeDtypeStruct((_NPAD, _H), jnp.float32),
               jax.ShapeDtypeStruct((_NPAD, _H), jnp.float32),
               jax.ShapeDtypeStruct((_NPAD, _EXT), jnp.float32),
               jax.ShapeDtypeStruct((_NPAD, _EXT), jnp.float32)],
)

_k4 = pl.pallas_call(
    _k4_body,
    grid=(_EGRID,),
    in_specs=[pl.BlockSpec((_EB, _EXT), lambda i: (i, 0)), _full_spec(8, _H)],
    out_specs=pl.BlockSpec((_EB // 128, 128), lambda i: (i, 0)),
    out_shape=jax.ShapeDtypeStruct((_EP_PAD // 128, 128), jnp.float32),
)


_MESH = plsc.VectorSubcoreMesh(core_axis_name="c", subcore_axis_name="s")


_SEG = 1792
_NSEG = _EPT // _SEG
_CCAP = 2304


def _segsum_body(h_hbm, sa_hbm, da_hbm, sb_hbm, db_hbm, z_hbm, outa, outb,
                 acc, sseg, dseg, csrc, cloc, rowbuf, sem, sem_s):
    c = lax.axis_index("c")
    s = lax.axis_index("s")
    zeros16 = jnp.zeros((16,), jnp.int32)
    dump16 = jnp.full((16,), _CHUNK, jnp.int32)

    def drain(nb):
        @pl.when(nb > 0)
        def _():
            pltpu.async_copy(h_hbm.at[csrc.at[pl.ds(0, 128)]], rowbuf.at[0], sem)

            def dbody(i, _):
                sl = lax.rem(i, 2)

                @pl.when(i + 1 < nb)
                def _():
                    pltpu.async_copy(
                        h_hbm.at[csrc.at[pl.ds((i + 1) * 128, 128)]],
                        rowbuf.at[1 - sl], sem)

                pltpu.make_async_copy(
                    h_hbm.at[csrc.at[pl.ds(i * 128, 128)]], rowbuf.at[sl], sem).wait()
                pltpu.async_copy(
                    rowbuf.at[sl], acc.at[cloc.at[pl.ds(i * 128, 128)]],
                    sem_s, add=True).wait()
                return 0

            lax.fori_loop(0, nb, dbody, 0)

    for src_hbm, dst_hbm, out_hbm in ((sa_hbm, da_hbm, outa), (sb_hbm, db_hbm, outb)):
        for k in range(_NCHUNK_PER_CORE):
            base = (_NCHUNK_PER_CORE * c + k) * _CHUNK
            pltpu.sync_copy(z_hbm, acc.at[pl.ds(s * _STRIP, _STRIP)])

            @pl.when(s == 0)
            def _():
                pltpu.sync_copy(z_hbm.at[pl.ds(0, 16)], acc.at[pl.ds(_CHUNK, 16)])

            plsc.subcore_barrier()

            def seg_body(g2, cofs):
                off_e = s * _EPT + g2 * _SEG
                pltpu.sync_copy(dst_hbm.at[pl.ds(off_e, _SEG)], dseg)
                pltpu.sync_copy(src_hbm.at[pl.ds(off_e, _SEG)], sseg)

                def grp(g, ofs):
                    d = dseg[pl.ds(g * 16, 16)]
                    sv = sseg[pl.ds(g * 16, 16)]
                    inb = (d >= base) & (d < base + _CHUNK)
                    m32 = jnp.where(inb, jnp.int32(1), jnp.int32(0))
                    pos = ofs + plsc.cumsum(m32) - 1
                    plsc.store_scatter(csrc, [pos], sv, mask=inb)
                    plsc.store_scatter(cloc, [pos], d - base, mask=inb)
                    return ofs + jnp.sum(m32)

                cofs = lax.fori_loop(0, _SEG // 16, grp, cofs)
                nfull = lax.div(cofs, 128)
                drain(nfull)
                rem = cofs - nfull * 128

                def mv(j, _):
                    vs = csrc[pl.ds(nfull * 128 + j * 16, 16)]
                    vl = cloc[pl.ds(nfull * 128 + j * 16, 16)]
                    csrc[pl.ds(j * 16, 16)] = vs
                    cloc[pl.ds(j * 16, 16)] = vl
                    return 0

                lax.fori_loop(0, lax.div(rem + 15, 16), mv, 0)
                return rem

            cofs = lax.fori_loop(0, _NSEG, seg_body, jnp.int32(0))

            def pad(j, _):
                csrc[pl.ds(cofs + j * 16, 16)] = zeros16
                cloc[pl.ds(cofs + j * 16, 16)] = dump16
                return 0

            lax.fori_loop(0, 8, pad, 0)
            drain(lax.div(cofs + 127, 128))
            plsc.subcore_barrier()
            for off, cnt in _STRIP_PIECES:
                pltpu.sync_copy(acc.at[pl.ds(s * _STRIP + off, cnt)],
                                out_hbm.at[pl.ds(base + s * _STRIP + off, cnt)])
            plsc.subcore_barrier()


_segsum = functools.partial(
    pl.kernel,
    out_type=(jax.ShapeDtypeStruct((_NPAD, _EXT), jnp.float32),
              jax.ShapeDtypeStruct((_NPAD, _EXT), jnp.float32)),
    mesh=_MESH,
    scratch_types=[
        pltpu.VMEM_SHARED((_ACC_ROWS, _EXT), jnp.float32),
        pltpu.VMEM((_SEG,), jnp.int32),
        pltpu.VMEM((_SEG,), jnp.int32),
        pltpu.VMEM((_CCAP,), jnp.int32),
        pltpu.VMEM((_CCAP,), jnp.int32),
        pltpu.VMEM((2, _SB, _EXT), jnp.float32),
        pltpu.SemaphoreType.DMA,
        pltpu.SemaphoreType.DMA,
    ],
    compiler_params=pltpu.CompilerParams(use_tc_tiling_on_sc=False,
                                         needs_layout_passes=False),
)(_segsum_body)


def _pairgather_body(p_hbm, q_hbm, ri_hbm, ci_hbm, out_hbm,
                     rowb, colb, bufp, bufq, semp, semq):
    c = lax.axis_index("c")
    s = lax.axis_index("s")
    wid = s * 2 + c
    base = wid * _EPTD

    def stage(b, sl):
        off_e = base + b * 128
        pltpu.sync_copy(ri_hbm.at[pl.ds(off_e, 128)], rowb.at[sl])
        pltpu.sync_copy(ci_hbm.at[pl.ds(off_e, 128)], colb.at[sl])
        pltpu.async_copy(p_hbm.at[rowb.at[sl]], bufp.at[sl], semp)
        pltpu.async_copy(q_hbm.at[colb.at[sl]], bufq.at[sl], semq)

    stage(0, 0)

    def batch(b, _):
        sl = lax.rem(b, 2)
        nsl = 1 - sl

        @pl.when(b + 1 < _NBD)
        def _():
            stage(b + 1, nsl)

        pltpu.make_async_copy(p_hbm.at[rowb.at[sl]], bufp.at[sl], semp).wait()
        pltpu.make_async_copy(q_hbm.at[colb.at[sl]], bufq.at[sl], semq).wait()

        def addrow(r, _):
            for g in range(_EXT // 16):
                bufp[sl, r, pl.ds(g * 16, 16)] = (
                    bufp[sl, r, pl.ds(g * 16, 16)] + bufq[sl, r, pl.ds(g * 16, 16)])
            return 0

        lax.fori_loop(0, 128, addrow, 0)
        pltpu.sync_copy(bufp.at[sl], out_hbm.at[pl.ds(base + b * 128, 128)])
        return 0

    lax.fori_loop(0, _NBD, batch, 0)


_pairgather = functools.partial(
    pl.kernel,
    out_type=jax.ShapeDtypeStruct((_EP_PAD, _EXT), jnp.float32),
    mesh=_MESH,
    scratch_types=[
        pltpu.VMEM((2, 128), jnp.int32),
        pltpu.VMEM((2, 128), jnp.int32),
        pltpu.VMEM((2, 128, _EXT), jnp.float32),
        pltpu.VMEM((2, 128, _EXT), jnp.float32),
        pltpu.SemaphoreType.DMA,
        pltpu.SemaphoreType.DMA,
    ],
    compiler_params=pltpu.CompilerParams(use_tc_tiling_on_sc=False),
)(_pairgather_body)


def kernel(x, edge_index_onset, edge_index_consecutive, pot_edges, pot_chord_edges,
           batch, onsets, durations, pitches, onset_beat, duration_beat, ts_beats,
           params):
    p = params
    npad = _NPAD - _N
    xp = jnp.pad(x, ((0, npad), (0, 0)))
    feat8 = jnp.pad(
        jnp.stack([onsets, durations, pitches, onset_beat, duration_beat,
                   ts_beats, jnp.zeros_like(onsets), jnp.zeros_like(onsets)], axis=1),
        ((0, npad), (0, 0)))

    def padi(a, val):
        return jnp.pad(a, (0, _E_PAD - a.shape[0]), constant_values=val)

    sa = padi(edge_index_onset[0], 0)
    da = padi(edge_index_onset[1], _N)
    sb = padi(edge_index_consecutive[0], 0)
    db = padi(edge_index_consecutive[1], _N)
    ri = padi(pot_edges[0], 0)
    ci = padi(pot_edges[1], 0)
    zsrc = jnp.zeros((_STRIP, _EXT), jnp.float32)

    r1 = lambda a: a.reshape(1, _H)
    fWT = p["first_W"].T
    fb = r1(p["first_b"])
    s00, s01 = p["sage"][0][0], p["sage"][0][1]
    s10, s11 = p["sage"][1][0], p["sage"][1][1]
    wl0_1, wl1_1 = 0.5 * s00["Wl"].T, 0.5 * s01["Wl"].T
    wrs_1 = 0.5 * (s00["Wr"].T + s01["Wr"].T)
    bs_1 = r1(0.5 * (s00["bl"] + s01["bl"]))
    wl0_2, wl1_2 = 0.5 * s10["Wl"].T, 0.5 * s11["Wl"].T
    wrs_2 = 0.5 * (s10["Wr"].T + s11["Wr"].T)
    bs_2 = r1(0.5 * (s10["bl"] + s11["bl"]))
    sw1 = p["staff_W1"].T
    sw2 = jnp.zeros((_H, _H), jnp.float32).at[:, :2].set(p["staff_W2"].T)
    sb2 = jnp.zeros((1, _H), jnp.float32).at[0, :2].set(p["staff_b2"])
    at_ = p["dec_W1"][:, :_H].T
    bt_ = p["dec_W1"][:, _H:2 * _H].T
    cvec = jnp.stack([
        p["dec_W1"][:, 2 * _H],
        p["dec_W1"][:, 2 * _H + 1],
        p["dec_W1"][:, 2 * _H + 2],
        p["dec_b1"],
        p["dec_ln_g"],
        p["dec_ln_b"],
        p["dec_W2"][0],
        jnp.full((_H,), p["dec_b2"][0], jnp.float32),
    ], axis=0)

    h_ext = _k1(xp, fWT, fb, r1(p["first_ln_g"]), r1(p["first_ln_b"]))
    sums_on, sums_co = _segsum(h_ext, sa, da, sb, db, zsrc)
    a, stats = _k2a(sums_on, sums_co, h_ext, wl0_1, wl1_1, wrs_1, bs_1)
    h1_ext = _k2b(a, stats, r1(p["gn_w"]), r1(p["gn_b"]), r1(p["gn_ms"]))
    sums1_on, sums1_co = _segsum(h1_ext, sa, da, sb, db, zsrc)
    hidden, staffpad, p_ext, q_ext = _k3(
        sums1_on, sums1_co, h1_ext, feat8, wl0_2, wl1_2, wrs_2, bs_2,
        sw1, r1(p["staff_b1"]), r1(p["staff_ln_g"]), r1(p["staff_ln_b"]),
        sw2, sb2, at_, bt_)
    rmat = _pairgather(p_ext, q_ext, ri, ci)
    outp = _k4(rmat, cvec)
    return (outp.reshape(-1)[:_EP], staffpad[:_N, :2], hidden[:_N])
